# baseline (speedup 1.0000x reference)
import functools

import jax
import jax.numpy as jnp
from jax import lax
from jax.experimental import pallas as pl
from jax.experimental.pallas import tpu as pltpu
from jax.experimental.pallas import tpu_sc as plsc

_NUM = 16
_ROW = 128


@functools.partial(jax.jit, static_argnums=(3, 4))
def _build_indices(dst_off, src_off, planes, n_workers, hw):
    p = planes.reshape(n_workers, -1)
    dst = p[:, :, None] * hw + dst_off[None, None, :]
    src = p[:, :, None] * hw + src_off[None, None, :]
    return (dst.reshape(n_workers, -1, _ROW).astype(jnp.int32),
            src.reshape(n_workers, -1, _ROW).astype(jnp.int32))


def _make_sc_kernel(n_elems, slab, rows):
    mesh = plsc.VectorSubcoreMesh(core_axis_name="c", subcore_axis_name="s")
    nc = plsc.get_sparse_core_info().num_cores

    @functools.partial(
        pl.kernel,
        out_type=jax.ShapeDtypeStruct((n_elems,), jnp.float32),
        mesh=mesh,
        scratch_types=[
            pltpu.VMEM((rows, _ROW), jnp.int32),
            pltpu.VMEM((rows, _ROW), jnp.int32),
            pltpu.VMEM((rows, _ROW), jnp.float32),
            pltpu.SemaphoreType.DMA,
            pltpu.SemaphoreType.DMA,
            pltpu.SemaphoreType.DMA,
        ],
    )
    def sc_kernel(x_hbm, dl_hbm, sl_hbm, out_hbm, dl_v, sl_v, vals_v,
                  csem, gsem, ssem):
        wid = lax.axis_index("s") * nc + lax.axis_index("c")
        base = wid * slab
        cp = pltpu.make_async_copy(
            x_hbm.at[pl.ds(base, slab)], out_hbm.at[pl.ds(base, slab)], csem)
        cp.start()
        pltpu.sync_copy(dl_hbm.at[wid], dl_v)
        pltpu.sync_copy(sl_hbm.at[wid], sl_v)
        gathers = []
        for j in range(rows):
            g = pltpu.make_async_copy(x_hbm.at[sl_v.at[j]], vals_v.at[j], gsem)
            g.start()
            gathers.append(g)
        for g in gathers:
            g.wait()
        cp.wait()
        scatters = []
        for j in range(rows):
            s = pltpu.make_async_copy(vals_v.at[j], out_hbm.at[dl_v.at[j]], ssem)
            s.start()
            scatters.append(s)
        for s in scatters:
            s.wait()

    return sc_kernel


def kernel(x):
    b, c, h, w = x.shape
    kx = jax.random.fold_in(jax.random.key(1), 0)
    ky = jax.random.fold_in(jax.random.key(1), 1)
    pool_x = jax.random.permutation(kx, h)
    pool_y = jax.random.permutation(ky, w)
    px = pool_x[:_NUM]
    py = pool_y[:_NUM]
    sx = pool_x[-_NUM:]
    sy = pool_y[-_NUM:]
    dst_off = (px * w + py).astype(jnp.int32)
    src_off = (sx * w + sy).astype(jnp.int32)

    p = b * c
    hw = h * w
    info = plsc.get_sparse_core_info()
    nw = info.num_cores * info.num_subcores
    assert p % nw == 0 and (p // nw) * _NUM % _ROW == 0
    slab = (p // nw) * hw
    rows = (p // nw) * _NUM // _ROW

    dst_lin, src_lin = _build_indices(dst_off, src_off,
                                      jnp.arange(p, dtype=jnp.int32), nw, hw)
    out = _make_sc_kernel(p * hw, slab, rows)(x.reshape(p * hw), dst_lin,
                                              src_lin)
    return out.reshape(b, c, h, w), (px, py)


# speedup vs baseline: 10.8194x; 10.8194x over previous
import functools

import jax
import jax.numpy as jnp
from jax import lax
from jax.experimental import pallas as pl
from jax.experimental.pallas import tpu as pltpu
from jax.experimental.pallas import tpu_sc as plsc

_NUM = 16


def _make_sc_kernel(n_planes, hw, planes_per_worker):
    mesh = plsc.VectorSubcoreMesh(core_axis_name="c", subcore_axis_name="s")
    nc = plsc.get_sparse_core_info().num_cores
    pw = planes_per_worker

    @functools.partial(
        pl.kernel,
        out_type=jax.ShapeDtypeStruct((n_planes, hw), jnp.float32),
        mesh=mesh,
        compiler_params=pltpu.CompilerParams(needs_layout_passes=False),
        scratch_types=[
            pltpu.VMEM((hw,), jnp.float32),
            pltpu.VMEM((hw,), jnp.float32),
            pltpu.VMEM((_NUM,), jnp.int32),
            pltpu.VMEM((_NUM,), jnp.int32),
            pltpu.SemaphoreType.DMA,
            pltpu.SemaphoreType.DMA,
            pltpu.SemaphoreType.DMA,
            pltpu.SemaphoreType.DMA,
        ],
    )
    def sc_kernel(x_hbm, doff_hbm, soff_hbm, out_hbm, buf0, buf1,
                  doff_v, soff_v, rsem0, rsem1, wsem0, wsem1):
        wid = lax.axis_index("s") * nc + lax.axis_index("c")
        base = wid * pw
        pltpu.sync_copy(doff_hbm, doff_v)
        pltpu.sync_copy(soff_hbm, soff_v)
        doff = doff_v[...]
        soff = soff_v[...]
        bufs = (buf0, buf1)
        rsems = (rsem0, rsem1)
        wsems = (wsem0, wsem1)
        reads = [None, None]
        writes = [None, None]
        for p in range(pw + 1):
            if p < pw:
                b = p % 2
                if writes[b] is not None:
                    writes[b].wait()
                reads[b] = pltpu.make_async_copy(
                    x_hbm.at[base + p], bufs[b], rsems[b])
                reads[b].start()
            if p >= 1:
                b = (p - 1) % 2
                reads[b].wait()
                vals = plsc.load_gather(bufs[b], [soff])
                plsc.store_scatter(bufs[b], [doff], vals)
                writes[b] = pltpu.make_async_copy(
                    bufs[b], out_hbm.at[base + p - 1], wsems[b])
                writes[b].start()
        writes[(pw - 1) % 2].wait()
        writes[pw % 2].wait()

    return sc_kernel


def kernel(x):
    b, c, h, w = x.shape
    kx = jax.random.fold_in(jax.random.key(1), 0)
    ky = jax.random.fold_in(jax.random.key(1), 1)
    pool_x = jax.random.permutation(kx, h)
    pool_y = jax.random.permutation(ky, w)
    px = pool_x[:_NUM]
    py = pool_y[:_NUM]
    sx = pool_x[-_NUM:]
    sy = pool_y[-_NUM:]
    dst_off = (px * w + py).astype(jnp.int32)
    src_off = (sx * w + sy).astype(jnp.int32)

    p = b * c
    hw = h * w
    info = plsc.get_sparse_core_info()
    nw = info.num_cores * info.num_subcores
    assert p % nw == 0
    out = _make_sc_kernel(p, hw, p // nw)(x.reshape(p, hw), dst_off, src_off)
    return out.reshape(b, c, h, w), (px, py)


# speedup vs baseline: 32.4041x; 2.9950x over previous
import functools

import jax
import jax.numpy as jnp
from jax import lax
from jax.experimental import pallas as pl
from jax.experimental.pallas import tpu as pltpu
from jax.experimental.pallas import tpu_sc as plsc

_NUM = 16


def _make_sc_kernel(b, c, h, w, planes_per_worker):
    mesh = plsc.VectorSubcoreMesh(core_axis_name="c", subcore_axis_name="s")
    nc = plsc.get_sparse_core_info().num_cores
    pw = planes_per_worker

    @functools.partial(
        pl.kernel,
        out_type=jax.ShapeDtypeStruct((b, c, h, w), jnp.float32),
        mesh=mesh,
        compiler_params=pltpu.CompilerParams(needs_layout_passes=False),
        scratch_types=[
            pltpu.VMEM((h, w), jnp.float32),
            pltpu.VMEM((h, w), jnp.float32),
            pltpu.VMEM((4, _NUM), jnp.int32),
            pltpu.SemaphoreType.DMA,
            pltpu.SemaphoreType.DMA,
            pltpu.SemaphoreType.DMA,
            pltpu.SemaphoreType.DMA,
        ],
    )
    def sc_kernel(x_hbm, idx_hbm, out_hbm, buf0, buf1, idx_v,
                  rsem0, rsem1, wsem0, wsem1):
        wid = lax.axis_index("s") * nc + lax.axis_index("c")
        base = wid * pw
        pltpu.sync_copy(idx_hbm, idx_v)
        pxv = idx_v[0, :]
        pyv = idx_v[1, :]
        sxv = idx_v[2, :]
        syv = idx_v[3, :]
        bufs = (buf0, buf1)
        rsems = (rsem0, rsem1)
        wsems = (wsem0, wsem1)
        reads = [None, None]
        writes = [None, None]
        for p in range(pw + 1):
            if p < pw:
                k = p % 2
                if writes[k] is not None:
                    writes[k].wait()
                pi = base + p
                reads[k] = pltpu.make_async_copy(
                    x_hbm.at[pi // c, pi % c], bufs[k], rsems[k])
                reads[k].start()
            if p >= 1:
                k = (p - 1) % 2
                reads[k].wait()
                vals = plsc.load_gather(bufs[k], [sxv, syv])
                plsc.store_scatter(bufs[k], [pxv, pyv], vals)
                pi = base + p - 1
                writes[k] = pltpu.make_async_copy(
                    bufs[k], out_hbm.at[pi // c, pi % c], wsems[k])
                writes[k].start()
        writes[(pw - 1) % 2].wait()
        writes[pw % 2].wait()

    return sc_kernel


def kernel(x):
    b, c, h, w = x.shape
    kx = jax.random.fold_in(jax.random.key(1), 0)
    ky = jax.random.fold_in(jax.random.key(1), 1)
    pool_x = jax.random.permutation(kx, h)
    pool_y = jax.random.permutation(ky, w)
    px = pool_x[:_NUM]
    py = pool_y[:_NUM]
    sx = pool_x[-_NUM:]
    sy = pool_y[-_NUM:]
    idx = jnp.stack([px, py, sx, sy]).astype(jnp.int32)

    p = b * c
    info = plsc.get_sparse_core_info()
    nw = info.num_cores * info.num_subcores
    assert p % nw == 0
    out = _make_sc_kernel(b, c, h, w, p // nw)(x, idx)
    return out, (px, py)


# speedup vs baseline: 34.9054x; 1.0772x over previous
import functools

import jax
import jax.numpy as jnp
import numpy as np
from jax import lax
from jax.experimental import pallas as pl
from jax.experimental.pallas import tpu as pltpu
from jax.experimental.pallas import tpu_sc as plsc

_NUM = 16


@functools.lru_cache(maxsize=None)
def _pools(h, w):
    with jax.ensure_compile_time_eval():
        with jax.default_device(jax.devices("cpu")[0]):
            kx = jax.random.fold_in(jax.random.key(1), 0)
            ky = jax.random.fold_in(jax.random.key(1), 1)
            pool_x = np.asarray(jax.random.permutation(kx, h))
            pool_y = np.asarray(jax.random.permutation(ky, w))
    return pool_x, pool_y


def _make_sc_kernel(b, c, h, w, planes_per_worker):
    mesh = plsc.VectorSubcoreMesh(core_axis_name="c", subcore_axis_name="s")
    nc = plsc.get_sparse_core_info().num_cores
    pw = planes_per_worker

    @functools.partial(
        pl.kernel,
        out_type=jax.ShapeDtypeStruct((b, c, h, w), jnp.float32),
        mesh=mesh,
        compiler_params=pltpu.CompilerParams(needs_layout_passes=False),
        scratch_types=[
            pltpu.VMEM((h, w), jnp.float32),
            pltpu.VMEM((h, w), jnp.float32),
            pltpu.VMEM((4, _NUM), jnp.int32),
            pltpu.SemaphoreType.DMA,
            pltpu.SemaphoreType.DMA,
            pltpu.SemaphoreType.DMA,
            pltpu.SemaphoreType.DMA,
        ],
    )
    def sc_kernel(x_hbm, idx_hbm, out_hbm, buf0, buf1, idx_v,
                  rsem0, rsem1, wsem0, wsem1):
        wid = lax.axis_index("s") * nc + lax.axis_index("c")
        base = wid * pw
        pltpu.sync_copy(idx_hbm, idx_v)
        pxv = idx_v[0, :]
        pyv = idx_v[1, :]
        sxv = idx_v[2, :]
        syv = idx_v[3, :]
        bufs = (buf0, buf1)
        rsems = (rsem0, rsem1)
        wsems = (wsem0, wsem1)
        reads = [None, None]
        writes = [None, None]
        for p in range(pw + 1):
            if p < pw:
                k = p % 2
                if writes[k] is not None:
                    writes[k].wait()
                pi = base + p
                reads[k] = pltpu.make_async_copy(
                    x_hbm.at[pi // c, pi % c], bufs[k], rsems[k])
                reads[k].start()
            if p >= 1:
                k = (p - 1) % 2
                reads[k].wait()
                vals = plsc.load_gather(bufs[k], [sxv, syv])
                plsc.store_scatter(bufs[k], [pxv, pyv], vals)
                pi = base + p - 1
                writes[k] = pltpu.make_async_copy(
                    bufs[k], out_hbm.at[pi // c, pi % c], wsems[k])
                writes[k].start()
        writes[(pw - 1) % 2].wait()
        writes[pw % 2].wait()

    return sc_kernel


def kernel(x):
    b, c, h, w = x.shape
    pool_x, pool_y = _pools(h, w)
    px = jnp.asarray(pool_x[:_NUM])
    py = jnp.asarray(pool_y[:_NUM])
    sx = pool_x[-_NUM:]
    sy = pool_y[-_NUM:]
    idx = jnp.asarray(
        np.stack([pool_x[:_NUM], pool_y[:_NUM], sx, sy]).astype(np.int32))

    p = b * c
    info = plsc.get_sparse_core_info()
    nw = info.num_cores * info.num_subcores
    assert p % nw == 0
    out = _make_sc_kernel(b, c, h, w, p // nw)(x, idx)
    return out, (px, py)
